# baseline (speedup 1.0000x reference)
import functools

import jax
import jax.numpy as jnp
from jax import lax
from jax.experimental import pallas as pl
from jax.experimental.pallas import tpu as pltpu
from jax.experimental.pallas import tpu_sc as plsc

_VOCAB = 1000000
_D = 64
_B = 16384
_NEG = 5

_NW = 32
_RW = _B // _NW
_CH = 128
_NCH = _RW // _CH
_NG = _CH // 16

_LN2 = 0.6931471805599453
_SQRT2 = 1.4142135623730951


def _log_f32(y):
    bits = plsc.bitcast(y, jnp.int32)
    e = ((bits >> 23) & 0xFF) - 127
    m = plsc.bitcast((bits & 0x7FFFFF) | (127 << 23), jnp.float32)
    big = m > _SQRT2
    m = jnp.where(big, m * 0.5, m)
    e = e + big.astype(jnp.int32)
    r = (m - 1.0) / (m + 1.0)
    r2 = r * r
    p = r2 * (1.0 / 9.0) + (1.0 / 7.0)
    p = p * r2 + (1.0 / 5.0)
    p = p * r2 + (1.0 / 3.0)
    p = p * r2 + 1.0
    return e.astype(jnp.float32) * _LN2 + 2.0 * r * p


def _softplus(x):
    return _log_f32(1.0 + jnp.exp(x))


def _sc_partials(pos_u, pos_v, neg_t, u_pairs, v_pairs):
    mesh = plsc.VectorSubcoreMesh(core_axis_name="c", subcore_axis_name="s")

    @functools.partial(
        pl.kernel,
        mesh=mesh,
        out_type=jax.ShapeDtypeStruct((_NW, 16), jnp.float32),
        compiler_params=pltpu.CompilerParams(needs_layout_passes=False),
        scratch_types=[
            pltpu.VMEM((_CH,), jnp.int32),
            pltpu.VMEM((_CH,), jnp.int32),
            pltpu.VMEM((_NEG, _CH), jnp.int32),
            pltpu.VMEM((_CH,), jnp.int32),
            pltpu.VMEM((_CH,), jnp.int32),
            pltpu.VMEM((_NEG, _CH), jnp.int32),
            pltpu.VMEM((_CH, 128), jnp.float32),
            pltpu.VMEM((_CH, 128), jnp.float32),
            pltpu.VMEM((_NEG * _CH, 128), jnp.float32),
            pltpu.VMEM((16,), jnp.float32),
            pltpu.SemaphoreType.DMA,
        ],
    )
    def k(pu_hbm, pv_hbm, nt_hbm, uw_hbm, vw_hbm, out_hbm,
          raw_u, raw_v, raw_n, pair_u, pair_v, pair_n,
          u_buf, v_buf, n_buf, accv, sem):
        wid = lax.axis_index("s") * 2 + lax.axis_index("c")
        row0 = wid * _RW

        def chunk_body(ci, acc):
            base = row0 + ci * _CH
            pltpu.sync_copy(pu_hbm.at[pl.ds(base, _CH)], raw_u)
            pltpu.sync_copy(pv_hbm.at[pl.ds(base, _CH)], raw_v)
            for j in range(_NEG):
                pltpu.sync_copy(nt_hbm.at[pl.ds(j, 1), pl.ds(base, _CH)],
                                raw_n.at[pl.ds(j, 1)])

            def _packed_row(r):
                return ((r >> 12) << 11) + (r & 2047)

            def halve_body(i, _):
                sl = pl.ds(i * 16, 16)
                pair_u[sl] = _packed_row(raw_u[sl])
                pair_v[sl] = _packed_row(raw_v[sl])
                for j in range(_NEG):
                    pair_n[j, sl] = _packed_row(raw_n[j, sl])
                return 0
            lax.fori_loop(0, _CH // 16, halve_body, 0)

            cp_u = pltpu.async_copy(uw_hbm.at[pair_u], u_buf, sem)
            cp_v = pltpu.async_copy(vw_hbm.at[pair_v], v_buf, sem)
            cps = [pltpu.async_copy(vw_hbm.at[pair_n.at[j]],
                                    n_buf.at[pl.ds(j * _CH, _CH)], sem)
                   for j in range(_NEG)]
            cp_u.wait()
            cp_v.wait()
            for cp in cps:
                cp.wait()

            def group_body(g, acc):
                rows = g * 16 + lax.iota(jnp.int32, 16)
                cb_u = ((raw_u[pl.ds(g * 16, 16)] >> 11) & 1) * 64
                cb_v = ((raw_v[pl.ds(g * 16, 16)] >> 11) & 1) * 64
                cb_n = [((raw_n[j, pl.ds(g * 16, 16)] >> 11) & 1) * 64
                        for j in range(_NEG)]
                pn = [rows + j * _CH for j in range(_NEG)]

                def d_body(d, carry):
                    sp, s0, s1, s2, s3, s4 = carry
                    du = plsc.load_gather(u_buf, [rows, cb_u + d])
                    dv = plsc.load_gather(v_buf, [rows, cb_v + d])
                    sp = sp + du * dv
                    n0 = plsc.load_gather(n_buf, [pn[0], cb_n[0] + d])
                    s0 = s0 + du * n0
                    n1 = plsc.load_gather(n_buf, [pn[1], cb_n[1] + d])
                    s1 = s1 + du * n1
                    n2 = plsc.load_gather(n_buf, [pn[2], cb_n[2] + d])
                    s2 = s2 + du * n2
                    n3 = plsc.load_gather(n_buf, [pn[3], cb_n[3] + d])
                    s3 = s3 + du * n3
                    n4 = plsc.load_gather(n_buf, [pn[4], cb_n[4] + d])
                    s4 = s4 + du * n4
                    return (sp, s0, s1, s2, s3, s4)

                z = jnp.zeros((16,), jnp.float32)
                sp, s0, s1, s2, s3, s4 = lax.fori_loop(
                    0, _D, d_body, (z, z, z, z, z, z))
                val = _softplus(-jnp.clip(sp, -10.0, 10.0))
                for sk in (s0, s1, s2, s3, s4):
                    val = val + _softplus(jnp.clip(sk, -10.0, 10.0))
                return acc + val

            return lax.fori_loop(0, _NG, group_body, acc)

        acc = lax.fori_loop(0, _NCH, chunk_body, jnp.zeros((16,), jnp.float32))
        accv[...] = acc
        pltpu.sync_copy(accv, out_hbm.at[wid])

    return k(pos_u, pos_v, neg_t, u_pairs, v_pairs)


_HV = _VOCAB // 2
_RW_BLK = 2048


def _repack(w_t):
    def body(in_ref, o_ref):
        x = in_ref[...]
        o_ref[:, 0:64] = x[:, :_RW_BLK].T
        o_ref[:, 64:128] = x[:, _RW_BLK:].T

    nj = pl.cdiv(_HV, _RW_BLK)
    return pl.pallas_call(
        body,
        grid=(nj,),
        in_specs=[pl.BlockSpec((64, 2 * _RW_BLK), lambda j: (0, j))],
        out_specs=pl.BlockSpec((_RW_BLK, 128), lambda j: (j, 0)),
        out_shape=jax.ShapeDtypeStruct((nj * _RW_BLK, 128), jnp.float32),
    )(w_t)


_NPB = 245
_CKL = 128


def _sc_repack(w_t):
    mesh = plsc.VectorSubcoreMesh(core_axis_name="c", subcore_axis_name="s")

    @functools.partial(
        pl.kernel,
        mesh=mesh,
        out_type=jax.ShapeDtypeStruct((_NPB * 2048, 128), jnp.float32),
        compiler_params=pltpu.CompilerParams(needs_layout_passes=False),
        scratch_types=[
            pltpu.VMEM((64, _CKL), jnp.float32),
            pltpu.VMEM((64, _CKL), jnp.float32),
            pltpu.VMEM((64, _CKL), jnp.float32),
            pltpu.VMEM((64, _CKL), jnp.float32),
            pltpu.VMEM((_CKL, 128), jnp.float32),
            pltpu.VMEM((_CKL, 128), jnp.float32),
            pltpu.SemaphoreType.DMA,
            pltpu.SemaphoreType.DMA,
            pltpu.SemaphoreType.DMA,
            pltpu.SemaphoreType.DMA,
            pltpu.SemaphoreType.DMA,
            pltpu.SemaphoreType.DMA,
        ],
    )
    def k(wt_hbm, out_hbm, ia0, ia1, ib0, ib1, os0, os1,
          sa0, sa1, sb0, sb1, so0, so1):
        wid = lax.axis_index("s") * 2 + lax.axis_index("c")

        def transpose_chunk(ib, os, cb, ncols):
            def r_body(r, _):
                rr = jnp.full((16,), 0, jnp.int32) + r
                for d0 in range(0, 64, 16):
                    dv = d0 + lax.iota(jnp.int32, 16)
                    val = plsc.load_gather(ib, [dv, rr])
                    plsc.store_scatter(os, [rr, dv + cb], val)
                return 0
            lax.fori_loop(0, ncols, r_body, 0)

        def issue_in(j, c, ia, ib, sa, sb):
            laneA = j * 4096 + c * _CKL
            cpa = pltpu.async_copy(wt_hbm.at[:, pl.ds(laneA, _CKL)], ia, sa)
            cpb = pltpu.async_copy(
                wt_hbm.at[:, pl.ds(laneA + 2048, _CKL)], ib, sb)
            return cpa, cpb

        def wait_in(ia, ib, sa, sb):
            pltpu.make_async_copy(wt_hbm.at[:, pl.ds(0, _CKL)], ia, sa).wait()
            pltpu.make_async_copy(wt_hbm.at[:, pl.ds(0, _CKL)], ib, sb).wait()

        def wait_out(os, so):
            pltpu.make_async_copy(
                os, out_hbm.at[pl.ds(0, _CKL), :], so).wait()

        def process_block(j):
            issue_in(j, 0, ia0, ib0, sa0, sb0)

            def c2_body(c2, _):
                c0 = 2 * c2
                wait_in(ia0, ib0, sa0, sb0)
                issue_in(j, c0 + 1, ia1, ib1, sa1, sb1)

                @pl.when(c2 > 0)
                def _():
                    wait_out(os0, so0)
                transpose_chunk(ia0, os0, 0, _CKL)
                transpose_chunk(ib0, os0, 64, _CKL)
                pltpu.async_copy(
                    os0,
                    out_hbm.at[pl.ds(j * 2048 + c0 * _CKL, _CKL), :], so0)

                wait_in(ia1, ib1, sa1, sb1)

                @pl.when(c2 < 7)
                def _():
                    issue_in(j, c0 + 2, ia0, ib0, sa0, sb0)

                @pl.when(c2 > 0)
                def _():
                    wait_out(os1, so1)
                transpose_chunk(ia1, os1, 0, _CKL)
                transpose_chunk(ib1, os1, 64, _CKL)
                pltpu.async_copy(
                    os1,
                    out_hbm.at[pl.ds(j * 2048 + (c0 + 1) * _CKL, _CKL), :],
                    so1)
                return 0

            lax.fori_loop(0, 8, c2_body, 0)
            wait_out(os0, so0)
            wait_out(os1, so1)

        def t_body(t, _):
            blk = wid + 32 * t

            @pl.when(blk < _NPB - 1)
            def _():
                process_block(blk)
            return 0

        lax.fori_loop(0, 8, t_body, 0)

    return _repack_tail(w_t, k(w_t))


def _repack_tail(w_t, packed):
    def body(in_ref, p_ref, o_ref):
        del p_ref
        x = in_ref[...]
        o_ref[:, 0:64] = x[:, :_RW_BLK].T
        o_ref[:, 64:128] = x[:, _RW_BLK:].T

    nj = pl.cdiv(_HV, _RW_BLK)
    return pl.pallas_call(
        body,
        grid=(1,),
        in_specs=[pl.BlockSpec((64, 2 * _RW_BLK), lambda j: (0, nj - 1)),
                  pl.BlockSpec(memory_space=pltpu.HBM)],
        out_specs=pl.BlockSpec((_RW_BLK, 128), lambda j: (nj - 1, 0)),
        out_shape=jax.ShapeDtypeStruct((nj * _RW_BLK, 128), jnp.float32),
        input_output_aliases={1: 0},
    )(w_t, packed)


def _finalize(partials):
    def body(p_ref, o_ref):
        o_ref[0, 0] = jnp.sum(p_ref[...]) * (1.0 / _B)

    out = pl.pallas_call(
        body,
        out_shape=jax.ShapeDtypeStruct((1, 1), jnp.float32),
        out_specs=pl.BlockSpec(memory_space=pltpu.SMEM),
    )(partials)
    return out[0, 0]


def kernel(pos_u, pos_v, neg_v, u_weight, v_weight):
    u_pairs = _sc_repack(u_weight.T)
    v_pairs = _repack(v_weight.T)
    neg_t = neg_v.astype(jnp.int32).T
    partials = _sc_partials(pos_u.astype(jnp.int32), pos_v.astype(jnp.int32),
                            neg_t, u_pairs, v_pairs)
    return _finalize(partials)


# speedup vs baseline: 2.1350x; 2.1350x over previous
import functools

import jax
import jax.numpy as jnp
from jax import lax
from jax.experimental import pallas as pl
from jax.experimental.pallas import tpu as pltpu
from jax.experimental.pallas import tpu_sc as plsc

_VOCAB = 1000000
_D = 64
_B = 16384
_NEG = 5

_NW = 32
_RW = _B // _NW
_CH = 128
_NCH = _RW // _CH
_NG = _CH // 16

_LN2 = 0.6931471805599453
_SQRT2 = 1.4142135623730951


def _log_f32(y):
    bits = plsc.bitcast(y, jnp.int32)
    e = ((bits >> 23) & 0xFF) - 127
    m = plsc.bitcast((bits & 0x7FFFFF) | (127 << 23), jnp.float32)
    big = m > _SQRT2
    m = jnp.where(big, m * 0.5, m)
    e = e + big.astype(jnp.int32)
    r = (m - 1.0) / (m + 1.0)
    r2 = r * r
    p = r2 * (1.0 / 9.0) + (1.0 / 7.0)
    p = p * r2 + (1.0 / 5.0)
    p = p * r2 + (1.0 / 3.0)
    p = p * r2 + 1.0
    return e.astype(jnp.float32) * _LN2 + 2.0 * r * p


def _softplus(x):
    return _log_f32(1.0 + jnp.exp(x))


def _sc_partials(pos_u, pos_v, neg_t, u_pairs, v_pairs):
    mesh = plsc.VectorSubcoreMesh(core_axis_name="c", subcore_axis_name="s")

    @functools.partial(
        pl.kernel,
        mesh=mesh,
        out_type=jax.ShapeDtypeStruct((_NW, 16), jnp.float32),
        compiler_params=pltpu.CompilerParams(needs_layout_passes=False),
        scratch_types=[
            pltpu.VMEM((_CH,), jnp.int32),
            pltpu.VMEM((_CH,), jnp.int32),
            pltpu.VMEM((_NEG, _CH), jnp.int32),
            pltpu.VMEM((_CH,), jnp.int32),
            pltpu.VMEM((_CH,), jnp.int32),
            pltpu.VMEM((_NEG, _CH), jnp.int32),
            pltpu.VMEM((_CH, 128), jnp.float32),
            pltpu.VMEM((_CH, 128), jnp.float32),
            pltpu.VMEM((_NEG * _CH, 128), jnp.float32),
            pltpu.VMEM((16,), jnp.float32),
            pltpu.SemaphoreType.DMA,
        ],
    )
    def k(pu_hbm, pv_hbm, nt_hbm, uw_hbm, vw_hbm, out_hbm,
          raw_u, raw_v, raw_n, pair_u, pair_v, pair_n,
          u_buf, v_buf, n_buf, accv, sem):
        wid = lax.axis_index("s") * 2 + lax.axis_index("c")
        row0 = wid * _RW

        def chunk_body(ci, acc):
            base = row0 + ci * _CH
            pltpu.sync_copy(pu_hbm.at[pl.ds(base, _CH)], raw_u)
            pltpu.sync_copy(pv_hbm.at[pl.ds(base, _CH)], raw_v)
            for j in range(_NEG):
                pltpu.sync_copy(nt_hbm.at[pl.ds(j, 1), pl.ds(base, _CH)],
                                raw_n.at[pl.ds(j, 1)])

            def _packed_row(r):
                return ((r >> 12) << 11) + (r & 2047)

            def halve_body(i, _):
                sl = pl.ds(i * 16, 16)
                pair_u[sl] = _packed_row(raw_u[sl])
                pair_v[sl] = _packed_row(raw_v[sl])
                for j in range(_NEG):
                    pair_n[j, sl] = _packed_row(raw_n[j, sl])
                return 0
            lax.fori_loop(0, _CH // 16, halve_body, 0)

            cp_u = pltpu.async_copy(uw_hbm.at[pair_u], u_buf, sem)
            cp_v = pltpu.async_copy(vw_hbm.at[pair_v], v_buf, sem)
            cps = [pltpu.async_copy(vw_hbm.at[pair_n.at[j]],
                                    n_buf.at[pl.ds(j * _CH, _CH)], sem)
                   for j in range(_NEG)]
            cp_u.wait()
            cp_v.wait()
            for cp in cps:
                cp.wait()

            def group_body(g, acc):
                rows = g * 16 + lax.iota(jnp.int32, 16)
                cb_u = ((raw_u[pl.ds(g * 16, 16)] >> 11) & 1) * 64
                cb_v = ((raw_v[pl.ds(g * 16, 16)] >> 11) & 1) * 64
                cb_n = [((raw_n[j, pl.ds(g * 16, 16)] >> 11) & 1) * 64
                        for j in range(_NEG)]
                pn = [rows + j * _CH for j in range(_NEG)]

                def d_body(d, carry):
                    sp, s0, s1, s2, s3, s4 = carry
                    du = plsc.load_gather(u_buf, [rows, cb_u + d])
                    dv = plsc.load_gather(v_buf, [rows, cb_v + d])
                    sp = sp + du * dv
                    n0 = plsc.load_gather(n_buf, [pn[0], cb_n[0] + d])
                    s0 = s0 + du * n0
                    n1 = plsc.load_gather(n_buf, [pn[1], cb_n[1] + d])
                    s1 = s1 + du * n1
                    n2 = plsc.load_gather(n_buf, [pn[2], cb_n[2] + d])
                    s2 = s2 + du * n2
                    n3 = plsc.load_gather(n_buf, [pn[3], cb_n[3] + d])
                    s3 = s3 + du * n3
                    n4 = plsc.load_gather(n_buf, [pn[4], cb_n[4] + d])
                    s4 = s4 + du * n4
                    return (sp, s0, s1, s2, s3, s4)

                z = jnp.zeros((16,), jnp.float32)
                sp, s0, s1, s2, s3, s4 = lax.fori_loop(
                    0, _D, d_body, (z, z, z, z, z, z))
                val = _softplus(-jnp.clip(sp, -10.0, 10.0))
                for sk in (s0, s1, s2, s3, s4):
                    val = val + _softplus(jnp.clip(sk, -10.0, 10.0))
                return acc + val

            return lax.fori_loop(0, _NG, group_body, acc)

        acc = lax.fori_loop(0, _NCH, chunk_body, jnp.zeros((16,), jnp.float32))
        accv[...] = acc
        pltpu.sync_copy(accv, out_hbm.at[wid])

    return k(pos_u, pos_v, neg_t, u_pairs, v_pairs)


_HV = _VOCAB // 2
_RW_BLK = 2048


def _repack(w_t):
    def body(in_ref, o_ref):
        x = in_ref[...]
        o_ref[:, 0:64] = x[:, :_RW_BLK].T
        o_ref[:, 64:128] = x[:, _RW_BLK:].T

    nj = pl.cdiv(_HV, _RW_BLK)
    return pl.pallas_call(
        body,
        grid=(nj,),
        in_specs=[pl.BlockSpec((64, 2 * _RW_BLK), lambda j: (0, j))],
        out_specs=pl.BlockSpec((_RW_BLK, 128), lambda j: (j, 0)),
        out_shape=jax.ShapeDtypeStruct((nj * _RW_BLK, 128), jnp.float32),
    )(w_t)


def _finalize(partials):
    def body(p_ref, o_ref):
        o_ref[0, 0] = jnp.sum(p_ref[...]) * (1.0 / _B)

    out = pl.pallas_call(
        body,
        out_shape=jax.ShapeDtypeStruct((1, 1), jnp.float32),
        out_specs=pl.BlockSpec(memory_space=pltpu.SMEM),
    )(partials)
    return out[0, 0]


def kernel(pos_u, pos_v, neg_v, u_weight, v_weight):
    u_pairs = _repack(u_weight.T)
    v_pairs = _repack(v_weight.T)
    neg_t = neg_v.astype(jnp.int32).T
    partials = _sc_partials(pos_u.astype(jnp.int32), pos_v.astype(jnp.int32),
                            neg_t, u_pairs, v_pairs)
    return _finalize(partials)


# speedup vs baseline: 2.5510x; 1.1949x over previous
import functools

import jax
import jax.numpy as jnp
from jax import lax
from jax.experimental import pallas as pl
from jax.experimental.pallas import tpu as pltpu
from jax.experimental.pallas import tpu_sc as plsc

_VOCAB = 1000000
_D = 64
_B = 16384
_NEG = 5

_NW = 32
_RW = _B // _NW
_CH = 128
_NCH = _RW // _CH
_NG = _CH // 16

_LN2 = 0.6931471805599453
_SQRT2 = 1.4142135623730951


def _log_f32(y):
    bits = plsc.bitcast(y, jnp.int32)
    e = ((bits >> 23) & 0xFF) - 127
    m = plsc.bitcast((bits & 0x7FFFFF) | (127 << 23), jnp.float32)
    big = m > _SQRT2
    m = jnp.where(big, m * 0.5, m)
    e = e + big.astype(jnp.int32)
    r = (m - 1.0) / (m + 1.0)
    r2 = r * r
    p = r2 * (1.0 / 9.0) + (1.0 / 7.0)
    p = p * r2 + (1.0 / 5.0)
    p = p * r2 + (1.0 / 3.0)
    p = p * r2 + 1.0
    return e.astype(jnp.float32) * _LN2 + 2.0 * r * p


def _softplus(x):
    return _log_f32(1.0 + jnp.exp(x))


def _sc_partials(pos_u, pos_v, neg_t, u_pairs, v_pairs):
    mesh = plsc.VectorSubcoreMesh(core_axis_name="c", subcore_axis_name="s")

    @functools.partial(
        pl.kernel,
        mesh=mesh,
        out_type=jax.ShapeDtypeStruct((_NW, 16), jnp.float32),
        compiler_params=pltpu.CompilerParams(needs_layout_passes=False),
        scratch_types=[
            pltpu.VMEM((_CH,), jnp.int32),
            pltpu.VMEM((_CH,), jnp.int32),
            pltpu.VMEM((_NEG, _CH), jnp.int32),
            pltpu.VMEM((_CH,), jnp.int32),
            pltpu.VMEM((_CH,), jnp.int32),
            pltpu.VMEM((_NEG, _CH), jnp.int32),
            pltpu.VMEM((_CH, 128), jnp.float32),
            pltpu.VMEM((_CH, 128), jnp.float32),
            pltpu.VMEM((_NEG * _CH, 128), jnp.float32),
            pltpu.VMEM((16,), jnp.float32),
            pltpu.SemaphoreType.DMA,
        ],
    )
    def k(pu_hbm, pv_hbm, nt_hbm, uw_hbm, vw_hbm, out_hbm,
          raw_u, raw_v, raw_n, pair_u, pair_v, pair_n,
          u_buf, v_buf, n_buf, accv, sem):
        wid = lax.axis_index("s") * 2 + lax.axis_index("c")
        row0 = wid * _RW

        def chunk_body(ci, acc):
            base = row0 + ci * _CH
            pltpu.sync_copy(pu_hbm.at[pl.ds(base, _CH)], raw_u)
            pltpu.sync_copy(pv_hbm.at[pl.ds(base, _CH)], raw_v)
            for j in range(_NEG):
                pltpu.sync_copy(nt_hbm.at[pl.ds(j, 1), pl.ds(base, _CH)],
                                raw_n.at[pl.ds(j, 1)])

            def _packed_row(r):
                return ((r >> (_SH + 1)) << _SH) + (r & (_W - 1))

            def halve_body(i, _):
                sl = pl.ds(i * 16, 16)
                pair_u[sl] = _packed_row(raw_u[sl])
                pair_v[sl] = _packed_row(raw_v[sl])
                for j in range(_NEG):
                    pair_n[j, sl] = _packed_row(raw_n[j, sl])
                return 0
            lax.fori_loop(0, _CH // 16, halve_body, 0)

            cp_u = pltpu.async_copy(uw_hbm.at[pair_u], u_buf, sem)
            cp_v = pltpu.async_copy(vw_hbm.at[pair_v], v_buf, sem)
            cps = [pltpu.async_copy(vw_hbm.at[pair_n.at[j]],
                                    n_buf.at[pl.ds(j * _CH, _CH)], sem)
                   for j in range(_NEG)]
            cp_u.wait()
            cp_v.wait()
            for cp in cps:
                cp.wait()

            def group_body(g, acc):
                rows = g * 16 + lax.iota(jnp.int32, 16)
                cb_u = ((raw_u[pl.ds(g * 16, 16)] >> _SH) & 1) * 64
                cb_v = ((raw_v[pl.ds(g * 16, 16)] >> _SH) & 1) * 64
                cb_n = [((raw_n[j, pl.ds(g * 16, 16)] >> _SH) & 1) * 64
                        for j in range(_NEG)]
                pn = [rows + j * _CH for j in range(_NEG)]

                def d_body(d, carry):
                    sp, s0, s1, s2, s3, s4 = carry
                    du = plsc.load_gather(u_buf, [rows, cb_u + d])
                    dv = plsc.load_gather(v_buf, [rows, cb_v + d])
                    sp = sp + du * dv
                    n0 = plsc.load_gather(n_buf, [pn[0], cb_n[0] + d])
                    s0 = s0 + du * n0
                    n1 = plsc.load_gather(n_buf, [pn[1], cb_n[1] + d])
                    s1 = s1 + du * n1
                    n2 = plsc.load_gather(n_buf, [pn[2], cb_n[2] + d])
                    s2 = s2 + du * n2
                    n3 = plsc.load_gather(n_buf, [pn[3], cb_n[3] + d])
                    s3 = s3 + du * n3
                    n4 = plsc.load_gather(n_buf, [pn[4], cb_n[4] + d])
                    s4 = s4 + du * n4
                    return (sp, s0, s1, s2, s3, s4)

                z = jnp.zeros((16,), jnp.float32)
                sp, s0, s1, s2, s3, s4 = lax.fori_loop(
                    0, _D, d_body, (z, z, z, z, z, z))
                val = _softplus(-jnp.clip(sp, -10.0, 10.0))
                for sk in (s0, s1, s2, s3, s4):
                    val = val + _softplus(jnp.clip(sk, -10.0, 10.0))
                return acc + val

            return lax.fori_loop(0, _NG, group_body, acc)

        acc = lax.fori_loop(0, _NCH, chunk_body, jnp.zeros((16,), jnp.float32))
        accv[...] = acc
        pltpu.sync_copy(accv, out_hbm.at[wid])

    return k(pos_u, pos_v, neg_t, u_pairs, v_pairs)


_HV = _VOCAB // 2
_RW_BLK = 4096
_W = _RW_BLK
_SH = _RW_BLK.bit_length() - 1


def _repack(w_t):
    def body(in_ref, o_ref):
        x = in_ref[...]
        o_ref[:, 0:64] = x[:, :_RW_BLK].T
        o_ref[:, 64:128] = x[:, _RW_BLK:].T

    nj = pl.cdiv(_HV, _RW_BLK)
    return pl.pallas_call(
        body,
        grid=(nj,),
        in_specs=[pl.BlockSpec((64, 2 * _RW_BLK), lambda j: (0, j))],
        out_specs=pl.BlockSpec((_RW_BLK, 128), lambda j: (j, 0)),
        out_shape=jax.ShapeDtypeStruct((nj * _RW_BLK, 128), jnp.float32),
    )(w_t)


def _finalize(partials):
    def body(p_ref, o_ref):
        o_ref[0, 0] = jnp.sum(p_ref[...]) * (1.0 / _B)

    out = pl.pallas_call(
        body,
        out_shape=jax.ShapeDtypeStruct((1, 1), jnp.float32),
        out_specs=pl.BlockSpec(memory_space=pltpu.SMEM),
    )(partials)
    return out[0, 0]


def kernel(pos_u, pos_v, neg_v, u_weight, v_weight):
    u_pairs = _repack(u_weight.T)
    v_pairs = _repack(v_weight.T)
    neg_t = neg_v.astype(jnp.int32).T
    partials = _sc_partials(pos_u.astype(jnp.int32), pos_v.astype(jnp.int32),
                            neg_t, u_pairs, v_pairs)
    return _finalize(partials)


# speedup vs baseline: 2.8139x; 1.1030x over previous
import functools

import jax
import jax.numpy as jnp
from jax import lax
from jax.experimental import pallas as pl
from jax.experimental.pallas import tpu as pltpu
from jax.experimental.pallas import tpu_sc as plsc

_VOCAB = 1000000
_D = 64
_B = 16384
_NEG = 5

_NW = 32
_RW = _B // _NW
_CH = 128
_NCH = _RW // _CH
_NG = _CH // 16

_LN2 = 0.6931471805599453
_SQRT2 = 1.4142135623730951


def _log_f32(y):
    bits = plsc.bitcast(y, jnp.int32)
    e = ((bits >> 23) & 0xFF) - 127
    m = plsc.bitcast((bits & 0x7FFFFF) | (127 << 23), jnp.float32)
    big = m > _SQRT2
    m = jnp.where(big, m * 0.5, m)
    e = e + big.astype(jnp.int32)
    r = (m - 1.0) / (m + 1.0)
    r2 = r * r
    p = r2 * (1.0 / 9.0) + (1.0 / 7.0)
    p = p * r2 + (1.0 / 5.0)
    p = p * r2 + (1.0 / 3.0)
    p = p * r2 + 1.0
    return e.astype(jnp.float32) * _LN2 + 2.0 * r * p


def _softplus(x):
    return _log_f32(1.0 + jnp.exp(x))


def _sc_partials(pos_u, pos_v, neg_t, u_pairs, v_pairs):
    mesh = plsc.VectorSubcoreMesh(core_axis_name="c", subcore_axis_name="s")

    @functools.partial(
        pl.kernel,
        mesh=mesh,
        out_type=jax.ShapeDtypeStruct((_NW, 16), jnp.float32),
        compiler_params=pltpu.CompilerParams(needs_layout_passes=False),
        scratch_types=[
            pltpu.VMEM((_CH,), jnp.int32),
            pltpu.VMEM((_CH,), jnp.int32),
            pltpu.VMEM((_NEG, _CH), jnp.int32),
            pltpu.VMEM((_CH,), jnp.int32),
            pltpu.VMEM((_CH,), jnp.int32),
            pltpu.VMEM((_NEG, _CH), jnp.int32),
            pltpu.VMEM((_CH, 128), jnp.float32),
            pltpu.VMEM((_CH, 128), jnp.float32),
            pltpu.VMEM((_NEG * _CH, 128), jnp.float32),
            pltpu.VMEM((16,), jnp.float32),
            pltpu.SemaphoreType.DMA,
        ],
    )
    def k(pu_hbm, pv_hbm, nt_hbm, uw_hbm, vw_hbm, out_hbm,
          raw_u, raw_v, raw_n, pair_u, pair_v, pair_n,
          u_buf, v_buf, n_buf, accv, sem):
        wid = lax.axis_index("s") * 2 + lax.axis_index("c")
        row0 = wid * _RW

        def chunk_body(ci, acc):
            base = row0 + ci * _CH
            pltpu.sync_copy(pu_hbm.at[pl.ds(base, _CH)], raw_u)
            pltpu.sync_copy(pv_hbm.at[pl.ds(base, _CH)], raw_v)
            for j in range(_NEG):
                pltpu.sync_copy(nt_hbm.at[pl.ds(j, 1), pl.ds(base, _CH)],
                                raw_n.at[pl.ds(j, 1)])

            def _packed_row(r):
                return ((r >> (_SH + 1)) << _SH) + (r & (_W - 1))

            def halve_body(i, _):
                sl = pl.ds(i * 16, 16)
                pair_u[sl] = _packed_row(raw_u[sl])
                pair_v[sl] = _packed_row(raw_v[sl])
                for j in range(_NEG):
                    pair_n[j, sl] = _packed_row(raw_n[j, sl])
                return 0
            lax.fori_loop(0, _CH // 16, halve_body, 0)

            cp_u = pltpu.async_copy(uw_hbm.at[pair_u], u_buf, sem)
            cp_v = pltpu.async_copy(vw_hbm.at[pair_v], v_buf, sem)
            cps = [pltpu.async_copy(vw_hbm.at[pair_n.at[j]],
                                    n_buf.at[pl.ds(j * _CH, _CH)], sem)
                   for j in range(_NEG)]
            cp_u.wait()
            cp_v.wait()
            for cp in cps:
                cp.wait()

            def group_body(g, acc):
                rows = g * 16 + lax.iota(jnp.int32, 16)
                cb_u = ((raw_u[pl.ds(g * 16, 16)] >> _SH) & 1) * 64
                cb_v = ((raw_v[pl.ds(g * 16, 16)] >> _SH) & 1) * 64
                cb_n = [((raw_n[j, pl.ds(g * 16, 16)] >> _SH) & 1) * 64
                        for j in range(_NEG)]
                pn = [rows + j * _CH for j in range(_NEG)]

                def d_body(d, carry):
                    sp, s0, s1, s2, s3, s4 = carry
                    du = plsc.load_gather(u_buf, [rows, cb_u + d])
                    dv = plsc.load_gather(v_buf, [rows, cb_v + d])
                    sp = sp + du * dv
                    n0 = plsc.load_gather(n_buf, [pn[0], cb_n[0] + d])
                    s0 = s0 + du * n0
                    n1 = plsc.load_gather(n_buf, [pn[1], cb_n[1] + d])
                    s1 = s1 + du * n1
                    n2 = plsc.load_gather(n_buf, [pn[2], cb_n[2] + d])
                    s2 = s2 + du * n2
                    n3 = plsc.load_gather(n_buf, [pn[3], cb_n[3] + d])
                    s3 = s3 + du * n3
                    n4 = plsc.load_gather(n_buf, [pn[4], cb_n[4] + d])
                    s4 = s4 + du * n4
                    return (sp, s0, s1, s2, s3, s4)

                z = jnp.zeros((16,), jnp.float32)
                sp, s0, s1, s2, s3, s4 = lax.fori_loop(
                    0, _D, d_body, (z, z, z, z, z, z))
                val = _softplus(-jnp.clip(sp, -10.0, 10.0))
                for sk in (s0, s1, s2, s3, s4):
                    val = val + _softplus(jnp.clip(sk, -10.0, 10.0))
                return acc + val

            return lax.fori_loop(0, _NG, group_body, acc)

        acc = lax.fori_loop(0, _NCH, chunk_body, jnp.zeros((16,), jnp.float32))
        accv[...] = acc
        pltpu.sync_copy(accv, out_hbm.at[wid])

    return k(pos_u, pos_v, neg_t, u_pairs, v_pairs)


_HV = _VOCAB // 2
_RW_BLK = 8192
_W = _RW_BLK
_SH = _RW_BLK.bit_length() - 1


def _repack(w_t):
    def body(in_ref, o_ref):
        x = in_ref[...]
        o_ref[:, 0:64] = x[:, :_RW_BLK].T
        o_ref[:, 64:128] = x[:, _RW_BLK:].T

    nj = pl.cdiv(_HV, _RW_BLK)
    return pl.pallas_call(
        body,
        grid=(nj,),
        in_specs=[pl.BlockSpec((64, 2 * _RW_BLK), lambda j: (0, j))],
        out_specs=pl.BlockSpec((_RW_BLK, 128), lambda j: (j, 0)),
        out_shape=jax.ShapeDtypeStruct((nj * _RW_BLK, 128), jnp.float32),
    )(w_t)


def _finalize(partials):
    def body(p_ref, o_ref):
        o_ref[0, 0] = jnp.sum(p_ref[...]) * (1.0 / _B)

    out = pl.pallas_call(
        body,
        out_shape=jax.ShapeDtypeStruct((1, 1), jnp.float32),
        out_specs=pl.BlockSpec(memory_space=pltpu.SMEM),
    )(partials)
    return out[0, 0]


def kernel(pos_u, pos_v, neg_v, u_weight, v_weight):
    u_pairs = _repack(u_weight.T)
    v_pairs = _repack(v_weight.T)
    neg_t = neg_v.astype(jnp.int32).T
    partials = _sc_partials(pos_u.astype(jnp.int32), pos_v.astype(jnp.int32),
                            neg_t, u_pairs, v_pairs)
    return _finalize(partials)


# speedup vs baseline: 2.9479x; 1.0476x over previous
import functools

import jax
import jax.numpy as jnp
from jax import lax
from jax.experimental import pallas as pl
from jax.experimental.pallas import tpu as pltpu
from jax.experimental.pallas import tpu_sc as plsc

_VOCAB = 1000000
_D = 64
_B = 16384
_NEG = 5

_NW = 32
_RW = _B // _NW
_CH = 128
_NCH = _RW // _CH
_NG = _CH // 16

_LN2 = 0.6931471805599453
_SQRT2 = 1.4142135623730951


def _log_f32(y):
    bits = plsc.bitcast(y, jnp.int32)
    e = ((bits >> 23) & 0xFF) - 127
    m = plsc.bitcast((bits & 0x7FFFFF) | (127 << 23), jnp.float32)
    big = m > _SQRT2
    m = jnp.where(big, m * 0.5, m)
    e = e + big.astype(jnp.int32)
    r = (m - 1.0) / (m + 1.0)
    r2 = r * r
    p = r2 * (1.0 / 9.0) + (1.0 / 7.0)
    p = p * r2 + (1.0 / 5.0)
    p = p * r2 + (1.0 / 3.0)
    p = p * r2 + 1.0
    return e.astype(jnp.float32) * _LN2 + 2.0 * r * p


def _softplus(x):
    return _log_f32(1.0 + jnp.exp(x))


def _sc_partials(pos_u, pos_v, neg_t, u_pairs, v_pairs):
    mesh = plsc.VectorSubcoreMesh(core_axis_name="c", subcore_axis_name="s")

    @functools.partial(
        pl.kernel,
        mesh=mesh,
        out_type=jax.ShapeDtypeStruct((_NW, 16), jnp.float32),
        compiler_params=pltpu.CompilerParams(needs_layout_passes=False),
        scratch_types=[
            pltpu.VMEM((_CH,), jnp.int32),
            pltpu.VMEM((_CH,), jnp.int32),
            pltpu.VMEM((_NEG, _CH), jnp.int32),
            pltpu.VMEM((_CH,), jnp.int32),
            pltpu.VMEM((_CH,), jnp.int32),
            pltpu.VMEM((_NEG, _CH), jnp.int32),
            pltpu.VMEM((_CH, 128), jnp.float32),
            pltpu.VMEM((_CH, 128), jnp.float32),
            pltpu.VMEM((_NEG * _CH, 128), jnp.float32),
            pltpu.VMEM((16,), jnp.float32),
            pltpu.SemaphoreType.DMA,
        ],
    )
    def k(pu_hbm, pv_hbm, nt_hbm, uw_hbm, vw_hbm, out_hbm,
          raw_u, raw_v, raw_n, pair_u, pair_v, pair_n,
          u_buf, v_buf, n_buf, accv, sem):
        wid = lax.axis_index("s") * 2 + lax.axis_index("c")
        row0 = wid * _RW

        def chunk_body(ci, acc):
            base = row0 + ci * _CH
            pltpu.sync_copy(pu_hbm.at[pl.ds(base, _CH)], raw_u)
            pltpu.sync_copy(pv_hbm.at[pl.ds(base, _CH)], raw_v)
            for j in range(_NEG):
                pltpu.sync_copy(nt_hbm.at[pl.ds(j, 1), pl.ds(base, _CH)],
                                raw_n.at[pl.ds(j, 1)])

            def _packed_row(r):
                return ((r >> (_SH + 1)) << _SH) + (r & (_W - 1))

            def halve_body(i, _):
                sl = pl.ds(i * 16, 16)
                pair_u[sl] = _packed_row(raw_u[sl])
                pair_v[sl] = _packed_row(raw_v[sl])
                for j in range(_NEG):
                    pair_n[j, sl] = _packed_row(raw_n[j, sl])
                return 0
            lax.fori_loop(0, _CH // 16, halve_body, 0)

            cp_u = pltpu.async_copy(uw_hbm.at[pair_u], u_buf, sem)
            cp_v = pltpu.async_copy(vw_hbm.at[pair_v], v_buf, sem)
            cps = [pltpu.async_copy(vw_hbm.at[pair_n.at[j]],
                                    n_buf.at[pl.ds(j * _CH, _CH)], sem)
                   for j in range(_NEG)]
            cp_u.wait()
            cp_v.wait()
            for cp in cps:
                cp.wait()

            def group_body(g, acc):
                rows = g * 16 + lax.iota(jnp.int32, 16)
                cb_u = ((raw_u[pl.ds(g * 16, 16)] >> _SH) & 1) * 64
                cb_v = ((raw_v[pl.ds(g * 16, 16)] >> _SH) & 1) * 64
                cb_n = [((raw_n[j, pl.ds(g * 16, 16)] >> _SH) & 1) * 64
                        for j in range(_NEG)]
                pn = [rows + j * _CH for j in range(_NEG)]

                def d_body(d, carry):
                    sp, s0, s1, s2, s3, s4 = carry
                    du = plsc.load_gather(u_buf, [rows, cb_u + d])
                    dv = plsc.load_gather(v_buf, [rows, cb_v + d])
                    sp = sp + du * dv
                    n0 = plsc.load_gather(n_buf, [pn[0], cb_n[0] + d])
                    s0 = s0 + du * n0
                    n1 = plsc.load_gather(n_buf, [pn[1], cb_n[1] + d])
                    s1 = s1 + du * n1
                    n2 = plsc.load_gather(n_buf, [pn[2], cb_n[2] + d])
                    s2 = s2 + du * n2
                    n3 = plsc.load_gather(n_buf, [pn[3], cb_n[3] + d])
                    s3 = s3 + du * n3
                    n4 = plsc.load_gather(n_buf, [pn[4], cb_n[4] + d])
                    s4 = s4 + du * n4
                    return (sp, s0, s1, s2, s3, s4)

                z = jnp.zeros((16,), jnp.float32)
                sp, s0, s1, s2, s3, s4 = lax.fori_loop(
                    0, _D, d_body, (z, z, z, z, z, z))
                val = _softplus(-jnp.clip(sp, -10.0, 10.0))
                for sk in (s0, s1, s2, s3, s4):
                    val = val + _softplus(jnp.clip(sk, -10.0, 10.0))
                return acc + val

            return lax.fori_loop(0, _NG, group_body, acc)

        acc = lax.fori_loop(0, _NCH, chunk_body, jnp.zeros((16,), jnp.float32))
        accv[...] = acc
        pltpu.sync_copy(accv, out_hbm.at[wid])

    return k(pos_u, pos_v, neg_t, u_pairs, v_pairs)


_HV = _VOCAB // 2
_RW_BLK = 16384
_W = _RW_BLK
_SH = _RW_BLK.bit_length() - 1


def _repack(w_t):
    def body(in_ref, o_ref):
        x = in_ref[...]
        o_ref[:, 0:64] = x[:, :_RW_BLK].T
        o_ref[:, 64:128] = x[:, _RW_BLK:].T

    nj = pl.cdiv(_HV, _RW_BLK)
    return pl.pallas_call(
        body,
        grid=(nj,),
        in_specs=[pl.BlockSpec((64, 2 * _RW_BLK), lambda j: (0, j))],
        out_specs=pl.BlockSpec((_RW_BLK, 128), lambda j: (j, 0)),
        out_shape=jax.ShapeDtypeStruct((nj * _RW_BLK, 128), jnp.float32),
    )(w_t)


def _finalize(partials):
    def body(p_ref, o_ref):
        o_ref[0, 0] = jnp.sum(p_ref[...]) * (1.0 / _B)

    out = pl.pallas_call(
        body,
        out_shape=jax.ShapeDtypeStruct((1, 1), jnp.float32),
        out_specs=pl.BlockSpec(memory_space=pltpu.SMEM),
    )(partials)
    return out[0, 0]


def kernel(pos_u, pos_v, neg_v, u_weight, v_weight):
    u_pairs = _repack(u_weight.T)
    v_pairs = _repack(v_weight.T)
    neg_t = neg_v.astype(jnp.int32).T
    partials = _sc_partials(pos_u.astype(jnp.int32), pos_v.astype(jnp.int32),
                            neg_t, u_pairs, v_pairs)
    return _finalize(partials)
